# baseline (speedup 1.0000x reference)
import functools
import math

import numpy as np
import jax
import jax.numpy as jnp
from jax import lax
from jax.experimental import pallas as pl
from jax.experimental.pallas import tpu as pltpu
from jax.experimental.pallas import tpu_sc as plsc

DIM = 1024
NUM_HEADS = 16
HEAD_DIM = 64
KN = 32
SEQ = 2048
CQ = 8
NW = 32
SCALE = 1.0 / math.sqrt(HEAD_DIM)


def _cantor_coords_np(seq_len: int, depth: int = 8) -> np.ndarray:
    pos = np.arange(seq_len, dtype=np.float32)
    x = pos / np.float32(max(1, seq_len - 1))
    x = np.clip(x, np.float32(1e-06), np.float32(1.0 - 1e-06)).astype(np.float32)
    val = np.zeros_like(x)
    factor = np.float32(0.5)
    for _ in range(depth):
        xs = (x * np.float32(3.0)).astype(np.float32)
        digit = np.floor(xs).astype(np.int32)
        x = (xs - digit.astype(np.float32)).astype(np.float32)
        val = (val + (digit == 2).astype(np.float32) * factor).astype(np.float32)
        factor = np.float32(factor * np.float32(0.5))
    return np.clip(val, np.float32(0.0), np.float32(1.0))


@functools.lru_cache(maxsize=None)
def _task_table(seq_len: int, k: int):
    coords = _cantor_coords_np(seq_len)
    dist = np.abs(coords[:, None] - coords[None, :])
    routes = np.argsort(dist, axis=-1, kind="stable")[:, :k].astype(np.int32)

    _, inv = np.unique(coords, return_inverse=True)
    ngroups = int(inv.max()) + 1
    task_kv, task_q = [], []
    for g in range(ngroups):
        members = np.where(inv == g)[0].astype(np.int32)
        rg = routes[members[0]]
        for c0 in range(0, len(members), CQ):
            chunk = members[c0:c0 + CQ]
            if len(chunk) < CQ:
                chunk = np.concatenate(
                    [chunk, np.full(CQ - len(chunk), chunk[0], np.int32)])
            task_kv.append(rg)
            task_q.append(chunk)
    task_kv = np.stack(task_kv).astype(np.int32)
    task_q = np.stack(task_q).astype(np.int32)
    return task_kv, task_q


_BM = 256
_BN = 256


def _qkv_body(x_ref, w_ref, b_ref, o_ref):
    s = pl.program_id(0)
    j = pl.program_id(2)
    acc = jnp.dot(x_ref[...], w_ref[...], preferred_element_type=jnp.float32)
    acc = acc + b_ref[s, pl.ds(j * _BN, _BN)][None, :]
    scale = jnp.where(s == 0, jnp.float32(SCALE), jnp.float32(1.0))
    o_ref[...] = (acc * scale)[None]


def _qkv_proj(x, Wqkv, bqkv):
    nj = DIM // _BN
    b2 = bqkv.reshape(3, DIM)
    return pl.pallas_call(
        _qkv_body,
        grid=(3, SEQ // _BM, nj),
        in_specs=[
            pl.BlockSpec((_BM, DIM), lambda s, i, j: (i, 0)),
            pl.BlockSpec((DIM, _BN), lambda s, i, j: (0, s * nj + j)),
            pl.BlockSpec((3, DIM), lambda s, i, j: (0, 0)),
        ],
        out_specs=pl.BlockSpec((1, _BM, _BN), lambda s, i, j: (s, i, j)),
        out_shape=jax.ShapeDtypeStruct((3, SEQ, DIM), jnp.float32),
    )(x, Wqkv, b2)


def _proj_body(x_ref, w_ref, b_ref, o_ref):
    acc = jnp.dot(x_ref[...], w_ref[...], preferred_element_type=jnp.float32)
    o_ref[...] = acc + b_ref[...]


def _out_proj(attn, Wout, bout):
    return pl.pallas_call(
        _proj_body,
        grid=(SEQ // _BM, DIM // _BN),
        in_specs=[
            pl.BlockSpec((_BM, DIM), lambda i, j: (i, 0)),
            pl.BlockSpec((DIM, _BN), lambda i, j: (0, j)),
            pl.BlockSpec((1, _BN), lambda i, j: (0, j)),
        ],
        out_specs=pl.BlockSpec((_BM, _BN), lambda i, j: (i, j)),
        out_shape=jax.ShapeDtypeStruct((SEQ, DIM), jnp.float32),
    )(attn, Wout, bout.reshape(1, DIM))


def _sc_attention(q, k, v, tkv, tq, n_tasks):
    mesh = plsc.VectorSubcoreMesh(core_axis_name="c", subcore_axis_name="s")
    niter = (n_tasks + NW - 1) // NW

    @functools.partial(
        pl.kernel,
        out_type=jax.ShapeDtypeStruct((SEQ, DIM), jnp.float32),
        mesh=mesh,
        compiler_params=pltpu.CompilerParams(
            use_tc_tiling_on_sc=False, needs_layout_passes=False),
        scratch_types=[
            pltpu.VMEM((KN, DIM), jnp.float32),
            pltpu.VMEM((KN, DIM), jnp.float32),
            pltpu.VMEM((CQ, DIM), jnp.float32),
            pltpu.VMEM((CQ, DIM), jnp.float32),
            pltpu.VMEM((CQ, KN), jnp.float32),
            pltpu.VMEM((KN,), jnp.int32),
            pltpu.VMEM((CQ,), jnp.int32),
            pltpu.SemaphoreType.DMA,
            pltpu.SemaphoreType.DMA,
            pltpu.SemaphoreType.DMA,
        ],
    )
    def kern(q_hbm, k_hbm, v_hbm, tkv_hbm, tq_hbm, out_hbm,
             kbuf, vbuf, qbuf, obuf, wbuf, kvidx, qidx, sem0, sem1, sem2):
        wid = lax.axis_index("s") * 2 + lax.axis_index("c")
        jv0 = lax.iota(jnp.int32, 16)
        jv1 = jv0 + 16

        def task_body(i, carry):
            t = i * NW + wid

            @pl.when(t < n_tasks)
            def _():
                pltpu.sync_copy(tkv_hbm.at[t], kvidx)
                pltpu.sync_copy(tq_hbm.at[t], qidx)
                cpk = pltpu.async_copy(k_hbm.at[kvidx], kbuf, sem0)
                cpv = pltpu.async_copy(v_hbm.at[kvidx], vbuf, sem1)
                cpq = pltpu.async_copy(q_hbm.at[qidx], qbuf, sem2)
                cpk.wait()
                cpv.wait()
                cpq.wait()

                def head_body(h, hcarry):
                    base = h * HEAD_DIM

                    def d_body(d, acc):
                        col = base + d
                        cvec = jnp.full((16,), col, jnp.int32)
                        kd0 = plsc.load_gather(kbuf, [jv0, cvec])
                        kd1 = plsc.load_gather(kbuf, [jv1, cvec])
                        new = []
                        for qi in range(CQ):
                            qs = plsc.load_gather(
                                qbuf, [jnp.full((16,), qi, jnp.int32), cvec])
                            new.append(acc[2 * qi] + kd0 * qs)
                            new.append(acc[2 * qi + 1] + kd1 * qs)
                        return tuple(new)

                    zero16 = jnp.zeros((16,), jnp.float32)
                    acc = lax.fori_loop(
                        0, HEAD_DIM, d_body,
                        tuple(zero16 for _ in range(2 * CQ)),
                        unroll=8)

                    for qi in range(CQ):
                        s0 = acc[2 * qi]
                        s1 = acc[2 * qi + 1]
                        m = jnp.maximum(jnp.max(s0), jnp.max(s1))
                        e0 = jnp.exp(s0 - m)
                        e1 = jnp.exp(s1 - m)
                        denom = jnp.full((16,), 1.0, jnp.float32) * (
                            jnp.sum(e0) + jnp.sum(e1))
                        r = jnp.full((16,), 1.0, jnp.float32) / denom
                        wbuf[qi, 0:16] = e0 * r
                        wbuf[qi, 16:32] = e1 * r

                    def j_body(j, oacc):
                        vj = [vbuf[j, pl.ds(base + 16 * c, 16)]
                              for c in range(4)]
                        jvec = jnp.full((16,), j, jnp.int32)
                        new = list(oacc)
                        for qi in range(CQ):
                            ws = plsc.load_gather(
                                wbuf, [jnp.full((16,), qi, jnp.int32), jvec])
                            for c in range(4):
                                new[4 * qi + c] = new[4 * qi + c] + vj[c] * ws
                        return tuple(new)

                    oacc = lax.fori_loop(
                        0, KN, j_body,
                        tuple(zero16 for _ in range(4 * CQ)),
                        unroll=8)
                    for qi in range(CQ):
                        for c in range(4):
                            obuf[qi, pl.ds(base + 16 * c, 16)] = oacc[4 * qi + c]
                    return hcarry

                lax.fori_loop(0, NUM_HEADS, head_body, 0)
                pltpu.async_copy(obuf, out_hbm.at[qidx], sem0).wait()

            return carry

        lax.fori_loop(0, niter, task_body, 0)

    return kern(q, k, v, tkv, tq)


def kernel(x, Wqkv, bqkv, Wout, bout):
    batch, seq_len, dim = x.shape
    tkv_np, tq_np = _task_table(seq_len, KN)
    n_tasks = tkv_np.shape[0]
    tkv = jnp.asarray(tkv_np)
    tq = jnp.asarray(tq_np)

    qkv = _qkv_proj(x.reshape(seq_len, dim), Wqkv, bqkv)
    q, k, v = qkv[0], qkv[1], qkv[2]
    attn = _sc_attention(q, k, v, tkv, tq, n_tasks)
    out = _out_proj(attn, Wout, bout)
    return out.reshape(batch, seq_len, dim)


# speedup vs baseline: 1.5908x; 1.5908x over previous
import functools
import math

import numpy as np
import jax
import jax.numpy as jnp
from jax import lax
from jax.experimental import pallas as pl
from jax.experimental.pallas import tpu as pltpu
from jax.experimental.pallas import tpu_sc as plsc

DIM = 1024
NUM_HEADS = 16
HEAD_DIM = 64
KN = 32
SEQ = 2048
CQ = 8
NW = 32
HALF = 512
HPH = 8
SCALE = 1.0 / math.sqrt(HEAD_DIM)


def _cantor_coords_np(seq_len: int, depth: int = 8) -> np.ndarray:
    pos = np.arange(seq_len, dtype=np.float32)
    x = pos / np.float32(max(1, seq_len - 1))
    x = np.clip(x, np.float32(1e-06), np.float32(1.0 - 1e-06)).astype(np.float32)
    val = np.zeros_like(x)
    factor = np.float32(0.5)
    for _ in range(depth):
        xs = (x * np.float32(3.0)).astype(np.float32)
        digit = np.floor(xs).astype(np.int32)
        x = (xs - digit.astype(np.float32)).astype(np.float32)
        val = (val + (digit == 2).astype(np.float32) * factor).astype(np.float32)
        factor = np.float32(factor * np.float32(0.5))
    return np.clip(val, np.float32(0.0), np.float32(1.0))


@functools.lru_cache(maxsize=None)
def _task_table(seq_len: int, k: int):
    coords = _cantor_coords_np(seq_len)
    dist = np.abs(coords[:, None] - coords[None, :])
    routes = np.argsort(dist, axis=-1, kind="stable")[:, :k].astype(np.int32)

    _, inv = np.unique(coords, return_inverse=True)
    ngroups = int(inv.max()) + 1
    task_kv, task_q = [], []
    for g in range(ngroups):
        members = np.where(inv == g)[0].astype(np.int32)
        rg = routes[members[0]]
        for c0 in range(0, len(members), CQ):
            chunk = members[c0:c0 + CQ]
            if len(chunk) < CQ:
                chunk = np.concatenate(
                    [chunk, np.full(CQ - len(chunk), chunk[0], np.int32)])
            task_kv.append(rg)
            task_q.append(chunk)
    task_kv = np.stack(task_kv).astype(np.int32)
    task_q = np.stack(task_q).astype(np.int32)
    return task_kv, task_q


_BM = 256
_BN = 256


def _qkv_body(x_ref, w_ref, b_ref, o_ref):
    s = pl.program_id(0)
    j = pl.program_id(2)
    acc = jnp.dot(x_ref[...], w_ref[...], preferred_element_type=jnp.float32)
    acc = acc + b_ref[s, pl.ds(j * _BN, _BN)][None, :]
    scale = jnp.where(s == 0, jnp.float32(SCALE), jnp.float32(1.0))
    o_ref[...] = (acc * scale)[None]


def _qkv_proj(x, Wqkv, bqkv):
    nj = DIM // _BN
    b2 = bqkv.reshape(3, DIM)
    return pl.pallas_call(
        _qkv_body,
        grid=(3, SEQ // _BM, nj),
        in_specs=[
            pl.BlockSpec((_BM, DIM), lambda s, i, j: (i, 0)),
            pl.BlockSpec((DIM, _BN), lambda s, i, j: (0, s * nj + j)),
            pl.BlockSpec((3, DIM), lambda s, i, j: (0, 0)),
        ],
        out_specs=pl.BlockSpec(
            (1, _BM, _BN), lambda s, i, j: (s * 2 + j // 2, i, j % 2)),
        out_shape=jax.ShapeDtypeStruct((6, SEQ, HALF), jnp.float32),
    )(x, Wqkv, b2)


def _proj_body(a0_ref, a1_ref, w_ref, b_ref, o_ref):
    acc = jnp.dot(a0_ref[...], w_ref[0:HALF, :],
                  preferred_element_type=jnp.float32)
    acc = acc + jnp.dot(a1_ref[...], w_ref[HALF:DIM, :],
                        preferred_element_type=jnp.float32)
    o_ref[...] = acc + b_ref[...]


def _out_proj(attn0, attn1, Wout, bout):
    return pl.pallas_call(
        _proj_body,
        grid=(SEQ // _BM, DIM // _BN),
        in_specs=[
            pl.BlockSpec((_BM, HALF), lambda i, j: (i, 0)),
            pl.BlockSpec((_BM, HALF), lambda i, j: (i, 0)),
            pl.BlockSpec((DIM, _BN), lambda i, j: (0, j)),
            pl.BlockSpec((1, _BN), lambda i, j: (0, j)),
        ],
        out_specs=pl.BlockSpec((_BM, _BN), lambda i, j: (i, j)),
        out_shape=jax.ShapeDtypeStruct((SEQ, DIM), jnp.float32),
    )(attn0, attn1, Wout, bout.reshape(1, DIM))


def _sc_attention(qkv6, tkv, tq, n_tasks):
    mesh = plsc.VectorSubcoreMesh(core_axis_name="c", subcore_axis_name="s")
    npairs = (n_tasks + NW - 1) // NW

    f32 = jnp.float32
    i32 = jnp.int32

    @functools.partial(
        pl.kernel,
        out_type=[jax.ShapeDtypeStruct((SEQ, HALF), f32),
                  jax.ShapeDtypeStruct((SEQ, HALF), f32)],
        mesh=mesh,
        compiler_params=pltpu.CompilerParams(
            use_tc_tiling_on_sc=False, needs_layout_passes=False),
        scratch_types=[
            pltpu.VMEM((KN, HALF), f32), pltpu.VMEM((KN, HALF), f32),
            pltpu.VMEM((KN, HALF), f32), pltpu.VMEM((KN, HALF), f32),
            pltpu.VMEM((CQ, HALF), f32), pltpu.VMEM((CQ, HALF), f32),
            pltpu.VMEM((CQ, HALF), f32), pltpu.VMEM((CQ, HALF), f32),
            pltpu.VMEM((CQ, KN), f32),
            pltpu.VMEM((KN,), i32), pltpu.VMEM((KN,), i32),
            pltpu.VMEM((CQ,), i32), pltpu.VMEM((CQ,), i32),
            pltpu.VMEM((CQ,), i32), pltpu.VMEM((CQ,), i32),
            pltpu.SemaphoreType.DMA, pltpu.SemaphoreType.DMA,
            pltpu.SemaphoreType.DMA, pltpu.SemaphoreType.DMA,
        ],
    )
    def kern(q0_h, q1_h, k0_h, k1_h, v0_h, v1_h, tkv_h, tq_h,
             out0_h, out1_h,
             kb0, kb1, vb0, vb1, qb0, qb1, ob0, ob1, wbuf,
             kvi0, kvi1, qi0, qi1, oqi0, oqi1, si0, si1, so0, so1):
        wid = lax.axis_index("s") * 2 + lax.axis_index("c")
        jv0 = lax.iota(i32, 16)
        jv1 = jv0 + 16

        q_h = (q0_h, q1_h)
        k_h = (k0_h, k1_h)
        v_h = (v0_h, v1_h)
        out_h = (out0_h, out1_h)
        kb = (kb0, kb1)
        vb = (vb0, vb1)
        qb = (qb0, qb1)
        ob = (ob0, ob1)
        kvi = (kvi0, kvi1)
        qi = (qi0, qi1)
        oqi = (oqi0, oqi1)
        si = (si0, si1)
        so = (so0, so1)

        ntask_me = (n_tasks - wid + NW - 1) // NW

        def issue_in(p, t):
            pltpu.sync_copy(tkv_h.at[t], kvi[p])
            pltpu.sync_copy(tq_h.at[t], qi[p])
            pltpu.async_copy(k_h[p].at[kvi[p]], kb[p], si[p])
            pltpu.async_copy(v_h[p].at[kvi[p]], vb[p], si[p])
            pltpu.async_copy(q_h[p].at[qi[p]], qb[p], si[p])

        def drain_in(p):
            pltpu.make_async_copy(k_h[p].at[kvi[p]], kb[p], si[p]).wait()
            pltpu.make_async_copy(v_h[p].at[kvi[p]], vb[p], si[p]).wait()
            pltpu.make_async_copy(q_h[p].at[qi[p]], qb[p], si[p]).wait()

        def drain_out(p):
            pltpu.make_async_copy(ob[p], out_h[p].at[oqi[p]], so[p]).wait()

        def compute(p):
            kbuf, vbuf, qbuf, obuf = kb[p], vb[p], qb[p], ob[p]

            def head_body(h, hcarry):
                base = h * HEAD_DIM

                def d_body(d, acc):
                    col = base + d
                    cvec = jnp.full((16,), col, i32)
                    kd0 = plsc.load_gather(kbuf, [jv0, cvec])
                    kd1 = plsc.load_gather(kbuf, [jv1, cvec])
                    new = []
                    for q_ in range(CQ):
                        qs = plsc.load_gather(
                            qbuf, [jnp.full((16,), q_, i32), cvec])
                        new.append(acc[2 * q_] + kd0 * qs)
                        new.append(acc[2 * q_ + 1] + kd1 * qs)
                    return tuple(new)

                zero16 = jnp.zeros((16,), f32)
                acc = lax.fori_loop(0, HEAD_DIM, d_body,
                                    tuple(zero16 for _ in range(2 * CQ)))

                ms = [jnp.maximum(jnp.max(acc[2 * q_]), jnp.max(acc[2 * q_ + 1]))
                      for q_ in range(CQ)]
                es = [(jnp.exp(acc[2 * q_] - ms[q_]),
                       jnp.exp(acc[2 * q_ + 1] - ms[q_]))
                      for q_ in range(CQ)]
                sums = [jnp.sum(e0) + jnp.sum(e1) for (e0, e1) in es]
                ones = jnp.full((16,), 1.0, f32)
                rs = [ones / (ones * sums[q_]) for q_ in range(CQ)]
                for q_ in range(CQ):
                    wbuf[q_, 0:16] = es[q_][0] * rs[q_]
                    wbuf[q_, 16:32] = es[q_][1] * rs[q_]

                def j_body(j, oacc):
                    vj = [vbuf[j, pl.ds(base + 16 * c, 16)] for c in range(4)]
                    jvec = jnp.full((16,), j, i32)
                    new = list(oacc)
                    for q_ in range(CQ):
                        ws = plsc.load_gather(
                            wbuf, [jnp.full((16,), q_, i32), jvec])
                        for c in range(4):
                            new[4 * q_ + c] = new[4 * q_ + c] + vj[c] * ws
                    return tuple(new)

                oacc = lax.fori_loop(0, KN, j_body,
                                     tuple(zero16 for _ in range(4 * CQ)))
                for q_ in range(CQ):
                    for c in range(4):
                        obuf[q_, pl.ds(base + 16 * c, 16)] = oacc[4 * q_ + c]
                return hcarry

            lax.fori_loop(0, HPH, head_body, 0)

        def pair_body(i2, carry):
            t = i2 * NW + wid
            tn = t + NW
            real = i2 < ntask_me

            @pl.when(real)
            def _():
                drain_in(0)

            @pl.when(real)
            def _():
                issue_in(1, t)

            @pl.when(real)
            def _():
                @pl.when(i2 >= 1)
                def _():
                    drain_out(0)
                compute(0)
                pltpu.sync_copy(tq_h.at[t], oqi[0])
                pltpu.async_copy(ob[0], out_h[0].at[oqi[0]], so[0])

            @pl.when(real)
            def _():
                drain_in(1)

            @pl.when(i2 + 1 < ntask_me)
            def _():
                issue_in(0, tn)

            @pl.when(real)
            def _():
                @pl.when(i2 >= 1)
                def _():
                    drain_out(1)
                compute(1)
                pltpu.sync_copy(tq_h.at[t], oqi[1])
                pltpu.async_copy(ob[1], out_h[1].at[oqi[1]], so[1])

            return carry

        issue_in(0, wid)
        lax.fori_loop(0, npairs, pair_body, 0)
        drain_out(0)
        drain_out(1)

    return kern(qkv6[0], qkv6[1], qkv6[2], qkv6[3], qkv6[4], qkv6[5], tkv, tq)


def kernel(x, Wqkv, bqkv, Wout, bout):
    batch, seq_len, dim = x.shape
    tkv_np, tq_np = _task_table(seq_len, KN)
    n_tasks = tkv_np.shape[0]
    tkv = jnp.asarray(tkv_np)
    tq = jnp.asarray(tq_np)

    qkv6 = _qkv_proj(x.reshape(seq_len, dim), Wqkv, bqkv)
    attn0, attn1 = _sc_attention(qkv6, tkv, tq, n_tasks)
    out = _out_proj(attn0, attn1, Wout, bout)
    return out.reshape(batch, seq_len, dim)


# speedup vs baseline: 1.9580x; 1.2308x over previous
import functools
import math

import numpy as np
import jax
import jax.numpy as jnp
from jax import lax
from jax.experimental import pallas as pl
from jax.experimental.pallas import tpu as pltpu
from jax.experimental.pallas import tpu_sc as plsc

DIM = 1024
NUM_HEADS = 16
HEAD_DIM = 64
KN = 32
SEQ = 2048
CQ = 8
NW = 32
HALF = 512
HPH = 8
SCALE = 1.0 / math.sqrt(HEAD_DIM)


def _cantor_coords_np(seq_len: int, depth: int = 8) -> np.ndarray:
    pos = np.arange(seq_len, dtype=np.float32)
    x = pos / np.float32(max(1, seq_len - 1))
    x = np.clip(x, np.float32(1e-06), np.float32(1.0 - 1e-06)).astype(np.float32)
    val = np.zeros_like(x)
    factor = np.float32(0.5)
    for _ in range(depth):
        xs = (x * np.float32(3.0)).astype(np.float32)
        digit = np.floor(xs).astype(np.int32)
        x = (xs - digit.astype(np.float32)).astype(np.float32)
        val = (val + (digit == 2).astype(np.float32) * factor).astype(np.float32)
        factor = np.float32(factor * np.float32(0.5))
    return np.clip(val, np.float32(0.0), np.float32(1.0))


@functools.lru_cache(maxsize=None)
def _task_table(seq_len: int, k: int):
    coords = _cantor_coords_np(seq_len)
    dist = np.abs(coords[:, None] - coords[None, :])
    routes = np.argsort(dist, axis=-1, kind="stable")[:, :k].astype(np.int32)

    _, inv = np.unique(coords, return_inverse=True)
    ngroups = int(inv.max()) + 1
    task_kv, task_q = [], []
    for g in range(ngroups):
        members = np.where(inv == g)[0].astype(np.int32)
        rg = routes[members[0]]
        for c0 in range(0, len(members), CQ):
            chunk = members[c0:c0 + CQ]
            if len(chunk) < CQ:
                chunk = np.concatenate(
                    [chunk, np.full(CQ - len(chunk), chunk[0], np.int32)])
            task_kv.append(rg)
            task_q.append(chunk)
    task_kv = np.stack(task_kv).astype(np.int32)
    task_q = np.stack(task_q).astype(np.int32)
    return task_kv, task_q


_BM = 256
_BN = 256


def _qkv_body(x_ref, w_ref, b_ref, o_ref):
    s = pl.program_id(0)
    j = pl.program_id(2)
    acc = jnp.dot(x_ref[...], w_ref[...], preferred_element_type=jnp.float32)
    acc = acc + b_ref[s, pl.ds(j * _BN, _BN)][None, :]
    scale = jnp.where(s == 0, jnp.float32(SCALE), jnp.float32(1.0))
    o_ref[...] = (acc * scale)[None]


def _qkv_proj(x, Wqkv, bqkv):
    nj = DIM // _BN
    b2 = bqkv.reshape(3, DIM)
    return pl.pallas_call(
        _qkv_body,
        grid=(3, SEQ // _BM, nj),
        in_specs=[
            pl.BlockSpec((_BM, DIM), lambda s, i, j: (i, 0)),
            pl.BlockSpec((DIM, _BN), lambda s, i, j: (0, s * nj + j)),
            pl.BlockSpec((3, DIM), lambda s, i, j: (0, 0)),
        ],
        out_specs=pl.BlockSpec(
            (1, _BM, _BN), lambda s, i, j: (s * 2 + j // 2, i, j % 2)),
        out_shape=jax.ShapeDtypeStruct((6, SEQ, HALF), jnp.float32),
    )(x, Wqkv, b2)


def _proj_body(a0_ref, a1_ref, w_ref, b_ref, o_ref):
    acc = jnp.dot(a0_ref[...], w_ref[0:HALF, :],
                  preferred_element_type=jnp.float32)
    acc = acc + jnp.dot(a1_ref[...], w_ref[HALF:DIM, :],
                        preferred_element_type=jnp.float32)
    o_ref[...] = acc + b_ref[...]


def _out_proj(attn0, attn1, Wout, bout):
    return pl.pallas_call(
        _proj_body,
        grid=(SEQ // _BM, DIM // _BN),
        in_specs=[
            pl.BlockSpec((_BM, HALF), lambda i, j: (i, 0)),
            pl.BlockSpec((_BM, HALF), lambda i, j: (i, 0)),
            pl.BlockSpec((DIM, _BN), lambda i, j: (0, j)),
            pl.BlockSpec((1, _BN), lambda i, j: (0, j)),
        ],
        out_specs=pl.BlockSpec((_BM, _BN), lambda i, j: (i, j)),
        out_shape=jax.ShapeDtypeStruct((SEQ, DIM), jnp.float32),
    )(attn0, attn1, Wout, bout.reshape(1, DIM))


def _sc_attention(qkv6, tkv, tq, n_tasks):
    mesh = plsc.VectorSubcoreMesh(core_axis_name="c", subcore_axis_name="s")
    npairs = (n_tasks + NW - 1) // NW

    f32 = jnp.float32
    i32 = jnp.int32

    @functools.partial(
        pl.kernel,
        out_type=[jax.ShapeDtypeStruct((SEQ, HALF), f32),
                  jax.ShapeDtypeStruct((SEQ, HALF), f32)],
        mesh=mesh,
        compiler_params=pltpu.CompilerParams(
            use_tc_tiling_on_sc=False, needs_layout_passes=False),
        scratch_types=[
            pltpu.VMEM((KN, HALF), f32), pltpu.VMEM((KN, HALF), f32),
            pltpu.VMEM((KN, HALF), f32), pltpu.VMEM((KN, HALF), f32),
            pltpu.VMEM((CQ, HALF), f32), pltpu.VMEM((CQ, HALF), f32),
            pltpu.VMEM((CQ, HALF), f32), pltpu.VMEM((CQ, HALF), f32),
            pltpu.VMEM((CQ, KN), f32),
            pltpu.VMEM((HALF * 33,), f32),
            pltpu.VMEM((KN,), i32), pltpu.VMEM((KN,), i32),
            pltpu.VMEM((CQ,), i32), pltpu.VMEM((CQ,), i32),
            pltpu.VMEM((CQ,), i32), pltpu.VMEM((CQ,), i32),
            pltpu.SemaphoreType.DMA, pltpu.SemaphoreType.DMA,
            pltpu.SemaphoreType.DMA, pltpu.SemaphoreType.DMA,
        ],
    )
    def kern(q0_h, q1_h, k0_h, k1_h, v0_h, v1_h, tkv_h, tq_h,
             out0_h, out1_h,
             kb0, kb1, vb0, vb1, qb0, qb1, ob0, ob1, wbuf, ktr,
             kvi0, kvi1, qi0, qi1, oqi0, oqi1, si0, si1, so0, so1):
        wid = lax.axis_index("s") * 2 + lax.axis_index("c")
        jv0 = lax.iota(i32, 16)
        jv1 = jv0 + 16

        q_h = (q0_h, q1_h)
        k_h = (k0_h, k1_h)
        v_h = (v0_h, v1_h)
        out_h = (out0_h, out1_h)
        kb = (kb0, kb1)
        vb = (vb0, vb1)
        qb = (qb0, qb1)
        ob = (ob0, ob1)
        kvi = (kvi0, kvi1)
        qi = (qi0, qi1)
        oqi = (oqi0, oqi1)
        si = (si0, si1)
        so = (so0, so1)

        ntask_me = (n_tasks - wid + NW - 1) // NW

        def issue_in(p, t):
            pltpu.sync_copy(tkv_h.at[t], kvi[p])
            pltpu.sync_copy(tq_h.at[t], qi[p])
            pltpu.async_copy(k_h[p].at[kvi[p]], kb[p], si[p])
            pltpu.async_copy(v_h[p].at[kvi[p]], vb[p], si[p])
            pltpu.async_copy(q_h[p].at[qi[p]], qb[p], si[p])

        def drain_in(p):
            pltpu.make_async_copy(k_h[p].at[kvi[p]], kb[p], si[p]).wait()
            pltpu.make_async_copy(v_h[p].at[kvi[p]], vb[p], si[p]).wait()
            pltpu.make_async_copy(q_h[p].at[qi[p]], qb[p], si[p]).wait()

        def drain_out(p):
            pltpu.make_async_copy(ob[p], out_h[p].at[oqi[p]], so[p]).wait()

        dl33 = lax.iota(i32, 16) * 33

        def compute(p):
            kbuf, vbuf, qbuf, obuf = kb[p], vb[p], qb[p], ob[p]

            def tr_body(j, carry):
                for c in range(KN):
                    vec = kbuf[j, pl.ds(c * 16, 16)]
                    plsc.store_scatter(ktr, [dl33 + (c * 528 + j)], vec)
                return carry

            lax.fori_loop(0, KN, tr_body, 0)

            def head_body(h, hcarry):
                base = h * HEAD_DIM

                def d_body(d, carry):
                    col33 = carry[0]
                    col = base + d
                    cvec = jnp.full((16,), col, i32)
                    kd0 = ktr[pl.ds(col33, 16)]
                    kd1 = ktr[pl.ds(col33 + 16, 16)]
                    acc = carry[1:]
                    new = [col33 + 33]
                    for q_ in range(CQ):
                        qs = plsc.load_gather(
                            qbuf, [jnp.full((16,), q_, i32), cvec])
                        new.append(acc[2 * q_] + kd0 * qs)
                        new.append(acc[2 * q_ + 1] + kd1 * qs)
                    return tuple(new)

                zero16 = jnp.zeros((16,), f32)
                carry = lax.fori_loop(
                    0, HEAD_DIM, d_body,
                    (base * 33,) + tuple(zero16 for _ in range(2 * CQ)))
                acc = carry[1:]

                ms = [jnp.maximum(jnp.max(acc[2 * q_]), jnp.max(acc[2 * q_ + 1]))
                      for q_ in range(CQ)]
                es = [(jnp.exp(acc[2 * q_] - ms[q_]),
                       jnp.exp(acc[2 * q_ + 1] - ms[q_]))
                      for q_ in range(CQ)]
                sums = [jnp.sum(e0) + jnp.sum(e1) for (e0, e1) in es]
                ones = jnp.full((16,), 1.0, f32)
                rs = [ones / (ones * sums[q_]) for q_ in range(CQ)]
                for q_ in range(CQ):
                    wbuf[q_, 0:16] = es[q_][0] * rs[q_]
                    wbuf[q_, 16:32] = es[q_][1] * rs[q_]

                def j_body(j, oacc):
                    vj = [vbuf[j, pl.ds(base + 16 * c, 16)] for c in range(4)]
                    jvec = jnp.full((16,), j, i32)
                    new = list(oacc)
                    for q_ in range(CQ):
                        ws = plsc.load_gather(
                            wbuf, [jnp.full((16,), q_, i32), jvec])
                        for c in range(4):
                            new[4 * q_ + c] = new[4 * q_ + c] + vj[c] * ws
                    return tuple(new)

                oacc = lax.fori_loop(0, KN, j_body,
                                     tuple(zero16 for _ in range(4 * CQ)))
                for q_ in range(CQ):
                    for c in range(4):
                        obuf[q_, pl.ds(base + 16 * c, 16)] = oacc[4 * q_ + c]
                return hcarry

            lax.fori_loop(0, HPH, head_body, 0)

        def pair_body(i2, carry):
            t = i2 * NW + wid
            tn = t + NW
            real = i2 < ntask_me

            @pl.when(real)
            def _():
                drain_in(0)

            @pl.when(real)
            def _():
                issue_in(1, t)

            @pl.when(real)
            def _():
                @pl.when(i2 >= 1)
                def _():
                    drain_out(0)
                compute(0)
                pltpu.sync_copy(tq_h.at[t], oqi[0])
                pltpu.async_copy(ob[0], out_h[0].at[oqi[0]], so[0])

            @pl.when(real)
            def _():
                drain_in(1)

            @pl.when(i2 + 1 < ntask_me)
            def _():
                issue_in(0, tn)

            @pl.when(real)
            def _():
                @pl.when(i2 >= 1)
                def _():
                    drain_out(1)
                compute(1)
                pltpu.sync_copy(tq_h.at[t], oqi[1])
                pltpu.async_copy(ob[1], out_h[1].at[oqi[1]], so[1])

            return carry

        issue_in(0, wid)
        lax.fori_loop(0, npairs, pair_body, 0)
        drain_out(0)
        drain_out(1)

    return kern(qkv6[0], qkv6[1], qkv6[2], qkv6[3], qkv6[4], qkv6[5], tkv, tq)


def kernel(x, Wqkv, bqkv, Wout, bout):
    batch, seq_len, dim = x.shape
    tkv_np, tq_np = _task_table(seq_len, KN)
    n_tasks = tkv_np.shape[0]
    tkv = jnp.asarray(tkv_np)
    tq = jnp.asarray(tq_np)

    qkv6 = _qkv_proj(x.reshape(seq_len, dim), Wqkv, bqkv)
    attn0, attn1 = _sc_attention(qkv6, tkv, tq, n_tasks)
    out = _out_proj(attn0, attn1, Wout, bout)
    return out.reshape(batch, seq_len, dim)
